# Initial kernel scaffold; baseline (speedup 1.0000x reference)
import jax
import jax.numpy as jnp
from jax.experimental import pallas as pl


def kernel(tl_reg, bl_reg, br_reg, tl_tag, bl_tag, br_tag, mask):
    raise NotImplementedError("write your pallas kernel here")


# speedup vs baseline: 2.2312x; 2.2312x over previous
import functools

import jax
import jax.numpy as jnp
from jax import lax
from jax.experimental import pallas as pl
from jax.experimental.pallas import tpu as pltpu
from jax.experimental.pallas import tpu_sc as plsc

_NC = 2
_NS = 16
_L = 16


def _rsqrt_newton(x):
    xi = lax.bitcast_convert_type(x, jnp.int32)
    yi = jnp.int32(0x5F3759DF) - lax.shift_right_logical(xi, 1)
    y = lax.bitcast_convert_type(yi, jnp.float32)
    half_x = 0.5 * x
    for _ in range(4):
        y = y * (1.5 - half_x * y * y)
    return y


def _sc_body(B, C, H, W, K,
             tl_hbm, bl_hbm, br_hbm, tlt_hbm, blt_hbm, brt_hbm, mask_hbm,
             out_hbm,
             tlt_v, blt_v, brt_v, mask_item_v,
             idx_tl, idx_bl, idx_br, val_tl, val_bl, val_br,
             out_v, sem):
    HW = H * W
    CHW = C * HW
    NW = _NC * _NS
    n_items = (B * K) // NW
    n_chunks = n_items // _L

    wid = lax.axis_index("s") * _NC + lax.axis_index("c")
    item_base = wid * n_items
    b = item_base // K
    half = (item_base // n_items) % (K // n_items)

    pltpu.sync_copy(tlt_hbm.at[pl.ds(item_base, n_items)], tlt_v)
    pltpu.sync_copy(blt_hbm.at[pl.ds(item_base, n_items)], blt_v)
    pltpu.sync_copy(brt_hbm.at[pl.ds(item_base, n_items)], brt_v)
    pltpu.sync_copy(mask_hbm.at[pl.ds(item_base, n_items)], mask_item_v)

    base_c0 = b * CHW
    for j in range(n_chunks):
        sl = pl.ds(j * _L, _L)
        sl_hi = pl.ds(n_items + j * _L, _L)
        t = tlt_v[sl] + base_c0
        idx_tl[sl] = t
        idx_tl[sl_hi] = t + HW
        t = blt_v[sl] + base_c0
        idx_bl[sl] = t
        idx_bl[sl_hi] = t + HW
        t = brt_v[sl] + base_c0
        idx_br[sl] = t
        idx_br[sl_hi] = t + HW

    c1 = pltpu.async_copy(tl_hbm.at[idx_tl], val_tl, sem)
    c2 = pltpu.async_copy(bl_hbm.at[idx_bl], val_bl, sem)
    c3 = pltpu.async_copy(br_hbm.at[idx_br], val_br, sem)
    c1.wait()
    c2.wait()
    c3.wait()

    if W & (W - 1) == 0:
        w_shift = W.bit_length() - 1

        def split_xy(t):
            return (t & (W - 1)).astype(jnp.float32), \
                   lax.shift_right_logical(t, w_shift).astype(jnp.float32)
    else:
        def split_xy(t):
            return (t % W).astype(jnp.float32), (t // W).astype(jnp.float32)

    acc = jnp.zeros((_L,), jnp.float32)
    for j in range(n_chunks):
        sl = pl.ds(j * _L, _L)
        sl_hi = pl.ds(n_items + j * _L, _L)
        tl_x, tl_y = split_xy(tlt_v[sl])
        bl_x, bl_y = split_xy(blt_v[sl])
        br_x, br_y = split_xy(brt_v[sl])
        tl_xs = tl_x + val_tl[sl]
        tl_ys = tl_y + val_tl[sl_hi]
        bl_xs = bl_x + val_bl[sl]
        bl_ys = bl_y + val_bl[sl_hi]
        br_xs = br_x + val_br[sl]
        br_ys = br_y + val_br[sl_hi]
        dx1 = bl_xs - tl_xs
        dy1 = bl_ys - tl_ys
        dx2 = bl_xs - br_xs
        dy2 = bl_ys - br_ys
        w2 = dx2 * dx2 + dy2 * dy2
        h2 = dx1 * dx1 + dy1 * dy1
        ip = dx1 * dx2 + dy1 * dy2
        contrib = jnp.abs(ip * _rsqrt_newton(w2 * h2)) * mask_item_v[sl]
        acc = acc + contrib

    out_v[...] = acc
    pltpu.sync_copy(out_v, out_hbm.at[b, pl.ds(half * _L, _L)])


def _tc_reduce_body(p_ref, m_ref, o_ref):
    num = jnp.sum(m_ref[...], axis=1, keepdims=True)
    bsum = jnp.sum(p_ref[...], axis=1, keepdims=True)
    o_ref[0, 0] = jnp.sum(bsum / (num + 0.0001))


def kernel(tl_reg, bl_reg, br_reg, tl_tag, bl_tag, br_tag, mask):
    B, C, H, W = tl_reg.shape
    K = tl_tag.shape[1]
    NW = _NC * _NS

    tl_f = tl_reg.reshape(-1)
    bl_f = bl_reg.reshape(-1)
    br_f = br_reg.reshape(-1)
    tlt = tl_tag.astype(jnp.int32).reshape(-1)
    blt = bl_tag.astype(jnp.int32).reshape(-1)
    brt = br_tag.astype(jnp.int32).reshape(-1)
    mask_f = mask.astype(jnp.float32).reshape(-1)

    n_items = (B * K) // NW
    mesh = plsc.VectorSubcoreMesh(core_axis_name="c", subcore_axis_name="s")
    sc = pl.kernel(
        functools.partial(_sc_body, B, C, H, W, K),
        out_type=jax.ShapeDtypeStruct((B, (NW // B) * _L), jnp.float32),
        mesh=mesh,
        scratch_types=[
            pltpu.VMEM((n_items,), jnp.int32),
            pltpu.VMEM((n_items,), jnp.int32),
            pltpu.VMEM((n_items,), jnp.int32),
            pltpu.VMEM((n_items,), jnp.float32),
            pltpu.VMEM((2 * n_items,), jnp.int32),
            pltpu.VMEM((2 * n_items,), jnp.int32),
            pltpu.VMEM((2 * n_items,), jnp.int32),
            pltpu.VMEM((2 * n_items,), jnp.float32),
            pltpu.VMEM((2 * n_items,), jnp.float32),
            pltpu.VMEM((2 * n_items,), jnp.float32),
            pltpu.VMEM((_L,), jnp.float32),
            pltpu.SemaphoreType.DMA,
        ],
    )
    partials = sc(tl_f, bl_f, br_f, tlt, blt, brt, mask_f)

    total = pl.pallas_call(
        _tc_reduce_body,
        out_shape=jax.ShapeDtypeStruct((1, 1), jnp.float32),
        out_specs=pl.BlockSpec(memory_space=pltpu.SMEM),
    )(partials, mask.astype(jnp.float32))
    return total[0, 0]


# speedup vs baseline: 2.4029x; 1.0770x over previous
import functools

import jax
import jax.numpy as jnp
from jax import lax
from jax.experimental import pallas as pl
from jax.experimental.pallas import tpu as pltpu
from jax.experimental.pallas import tpu_sc as plsc

_NS = 16
_L = 16


def _rsqrt_newton(x):
    xi = lax.bitcast_convert_type(x, jnp.int32)
    yi = jnp.int32(0x5F3759DF) - lax.shift_right_logical(xi, 1)
    y = lax.bitcast_convert_type(yi, jnp.float32)
    half_x = 0.5 * x
    for _ in range(4):
        y = y * (1.5 - half_x * y * y)
    return y


def _sc_body(B, C, H, W, K,
             tl_hbm, bl_hbm, br_hbm, tlt_hbm, blt_hbm, brt_hbm, mask_hbm,
             out_hbm,
             tlt_v, blt_v, brt_v, mask_v,
             idx_tl0, idx_tl1, idx_bl0, idx_bl1, idx_br0, idx_br1,
             val_tl0, val_tl1, val_bl0, val_bl1, val_br0, val_br1,
             src_sc, idx_sc, loss_v, zeros_v, lidx, sn_v, res_v,
             shared, sem):
    HW = H * W
    CHW = C * HW
    n_chunks = K // _L

    b = lax.axis_index("s")
    item_base = b * K

    pltpu.sync_copy(tlt_hbm.at[pl.ds(item_base, K)], tlt_v)
    pltpu.sync_copy(blt_hbm.at[pl.ds(item_base, K)], blt_v)
    pltpu.sync_copy(brt_hbm.at[pl.ds(item_base, K)], brt_v)
    pltpu.sync_copy(mask_hbm.at[pl.ds(item_base, K)], mask_v)

    base_c0 = b * CHW
    for j in range(n_chunks):
        sl = pl.ds(j * _L, _L)
        t = tlt_v[sl] + base_c0
        idx_tl0[sl] = t
        idx_tl1[sl] = t + HW
        t = blt_v[sl] + base_c0
        idx_bl0[sl] = t
        idx_bl1[sl] = t + HW
        t = brt_v[sl] + base_c0
        idx_br0[sl] = t
        idx_br1[sl] = t + HW

    copies = [
        pltpu.async_copy(tl_hbm.at[idx_tl0], val_tl0, sem),
        pltpu.async_copy(tl_hbm.at[idx_tl1], val_tl1, sem),
        pltpu.async_copy(bl_hbm.at[idx_bl0], val_bl0, sem),
        pltpu.async_copy(bl_hbm.at[idx_bl1], val_bl1, sem),
        pltpu.async_copy(br_hbm.at[idx_br0], val_br0, sem),
        pltpu.async_copy(br_hbm.at[idx_br1], val_br1, sem),
    ]
    for c in copies:
        c.wait()

    if W & (W - 1) == 0:
        w_shift = W.bit_length() - 1

        def split_xy(t):
            return (t & (W - 1)).astype(jnp.float32), \
                   lax.shift_right_logical(t, w_shift).astype(jnp.float32)
    else:
        def split_xy(t):
            return (t % W).astype(jnp.float32), (t // W).astype(jnp.float32)

    acc = jnp.zeros((_L,), jnp.float32)
    n_acc = jnp.zeros((_L,), jnp.float32)
    for j in range(n_chunks):
        sl = pl.ds(j * _L, _L)
        tl_x, tl_y = split_xy(tlt_v[sl])
        bl_x, bl_y = split_xy(blt_v[sl])
        br_x, br_y = split_xy(brt_v[sl])
        tl_xs = tl_x + val_tl0[sl]
        tl_ys = tl_y + val_tl1[sl]
        bl_xs = bl_x + val_bl0[sl]
        bl_ys = bl_y + val_bl1[sl]
        br_xs = br_x + val_br0[sl]
        br_ys = br_y + val_br1[sl]
        dx1 = bl_xs - tl_xs
        dy1 = bl_ys - tl_ys
        dx2 = bl_xs - br_xs
        dy2 = bl_ys - br_ys
        w2 = dx2 * dx2 + dy2 * dy2
        h2 = dx1 * dx1 + dy1 * dy1
        ip = dx1 * dx2 + dy1 * dy2
        m = mask_v[sl]
        acc = acc + jnp.abs(ip * _rsqrt_newton(w2 * h2)) * m
        n_acc = n_acc + m

    src_sc[pl.ds(0, _L)] = acc
    src_sc[pl.ds(_L, _L)] = n_acc
    idx_sc[pl.ds(0, _L)] = jnp.full((_L,), b, jnp.int32)
    idx_sc[pl.ds(_L, _L)] = jnp.full((_L,), _L + b, jnp.int32)
    zeros_v[pl.ds(0, _L)] = jnp.zeros((_L,), jnp.float32)
    zeros_v[pl.ds(_L, _L)] = jnp.zeros((_L,), jnp.float32)
    zeros_v[pl.ds(2 * _L, _L)] = jnp.zeros((_L,), jnp.float32)

    @pl.when(b == 0)
    def _():
        pltpu.sync_copy(zeros_v, shared)

    plsc.subcore_barrier()
    pltpu.sync_copy(src_sc, shared.at[idx_sc], add=True)
    plsc.subcore_barrier()

    @pl.when(b == 0)
    def _():
        pltpu.sync_copy(shared, sn_v)
        s_vec = sn_v[pl.ds(0, _L)]
        n_vec = sn_v[pl.ds(_L, _L)]
        loss_v[...] = s_vec / (n_vec + 0.0001)
        lidx[...] = jnp.full((_L,), 2 * _L, jnp.int32)
        pltpu.sync_copy(loss_v, shared.at[lidx], add=True)
        pltpu.sync_copy(shared.at[pl.ds(2 * _L, _L)], res_v)
        pltpu.sync_copy(res_v, out_hbm)


def kernel(tl_reg, bl_reg, br_reg, tl_tag, bl_tag, br_tag, mask):
    B, C, H, W = tl_reg.shape
    K = tl_tag.shape[1]

    tl_f = tl_reg.reshape(-1)
    bl_f = bl_reg.reshape(-1)
    br_f = br_reg.reshape(-1)
    tlt = tl_tag.astype(jnp.int32).reshape(-1)
    blt = bl_tag.astype(jnp.int32).reshape(-1)
    brt = br_tag.astype(jnp.int32).reshape(-1)
    mask_f = mask.astype(jnp.float32).reshape(-1)

    mesh = plsc.VectorSubcoreMesh(core_axis_name="c", subcore_axis_name="s",
                                  num_cores=1, num_subcores=_NS)
    sc = pl.kernel(
        functools.partial(_sc_body, B, C, H, W, K),
        out_type=jax.ShapeDtypeStruct((_L,), jnp.float32),
        mesh=mesh,
        scratch_types=[
            pltpu.VMEM((K,), jnp.int32),
            pltpu.VMEM((K,), jnp.int32),
            pltpu.VMEM((K,), jnp.int32),
            pltpu.VMEM((K,), jnp.float32),
            pltpu.VMEM((K,), jnp.int32),
            pltpu.VMEM((K,), jnp.int32),
            pltpu.VMEM((K,), jnp.int32),
            pltpu.VMEM((K,), jnp.int32),
            pltpu.VMEM((K,), jnp.int32),
            pltpu.VMEM((K,), jnp.int32),
            pltpu.VMEM((K,), jnp.float32),
            pltpu.VMEM((K,), jnp.float32),
            pltpu.VMEM((K,), jnp.float32),
            pltpu.VMEM((K,), jnp.float32),
            pltpu.VMEM((K,), jnp.float32),
            pltpu.VMEM((K,), jnp.float32),
            pltpu.VMEM((2 * _L,), jnp.float32),
            pltpu.VMEM((2 * _L,), jnp.int32),
            pltpu.VMEM((_L,), jnp.float32),
            pltpu.VMEM((3 * _L,), jnp.float32),
            pltpu.VMEM((_L,), jnp.int32),
            pltpu.VMEM((3 * _L,), jnp.float32),
            pltpu.VMEM((_L,), jnp.float32),
            pltpu.VMEM_SHARED((3 * _L,), jnp.float32),
            pltpu.SemaphoreType.DMA,
        ],
    )
    out = sc(tl_f, bl_f, br_f, tlt, blt, brt, mask_f)
    return out[0]


# speedup vs baseline: 2.4212x; 1.0076x over previous
import functools

import jax
import jax.numpy as jnp
from jax import lax
from jax.experimental import pallas as pl
from jax.experimental.pallas import tpu as pltpu
from jax.experimental.pallas import tpu_sc as plsc

_NS = 16
_L = 16


def _rsqrt_newton(x):
    xi = lax.bitcast_convert_type(x, jnp.int32)
    yi = jnp.int32(0x5F3759DF) - lax.shift_right_logical(xi, 1)
    y = lax.bitcast_convert_type(yi, jnp.float32)
    half_x = 0.5 * x
    for _ in range(4):
        y = y * (1.5 - half_x * y * y)
    return y


def _sc_body(B, C, H, W, K, mshift,
             tl_hbm, bl_hbm, br_hbm, tlt_hbm, blt_hbm, brt_hbm,
             out_hbm,
             tlt_v, blt_v, brt_v,
             idx_tl0, idx_tl1, idx_bl0, idx_bl1, idx_br0, idx_br1,
             val_tl0, val_tl1, val_bl0, val_bl1, val_br0, val_br1,
             src_sc, idx_sc, loss_v, zeros_v, lidx, sn_v, res_v,
             shared, sem):
    HW = H * W
    CHW = C * HW
    n_chunks = K // _L
    tag_mask = (1 << mshift) - 1

    b = lax.axis_index("s")
    item_base = b * K

    pltpu.sync_copy(tlt_hbm.at[pl.ds(item_base, K)], tlt_v)
    pltpu.sync_copy(blt_hbm.at[pl.ds(item_base, K)], blt_v)
    pltpu.sync_copy(brt_hbm.at[pl.ds(item_base, K)], brt_v)

    base_c0 = b * CHW
    for j in range(n_chunks):
        sl = pl.ds(j * _L, _L)
        t = (tlt_v[sl] & tag_mask) + base_c0
        idx_tl0[sl] = t
        idx_tl1[sl] = t + HW
        t = blt_v[sl] + base_c0
        idx_bl0[sl] = t
        idx_bl1[sl] = t + HW
        t = brt_v[sl] + base_c0
        idx_br0[sl] = t
        idx_br1[sl] = t + HW

    copies = [
        pltpu.async_copy(tl_hbm.at[idx_tl0], val_tl0, sem),
        pltpu.async_copy(tl_hbm.at[idx_tl1], val_tl1, sem),
        pltpu.async_copy(bl_hbm.at[idx_bl0], val_bl0, sem),
        pltpu.async_copy(bl_hbm.at[idx_bl1], val_bl1, sem),
        pltpu.async_copy(br_hbm.at[idx_br0], val_br0, sem),
        pltpu.async_copy(br_hbm.at[idx_br1], val_br1, sem),
    ]
    for c in copies:
        c.wait()

    if W & (W - 1) == 0:
        w_shift = W.bit_length() - 1

        def split_xy(t):
            return (t & (W - 1)).astype(jnp.float32), \
                   lax.shift_right_logical(t, w_shift).astype(jnp.float32)
    else:
        def split_xy(t):
            return (t % W).astype(jnp.float32), (t // W).astype(jnp.float32)

    acc = jnp.zeros((_L,), jnp.float32)
    n_acc = jnp.zeros((_L,), jnp.float32)
    for j in range(n_chunks):
        sl = pl.ds(j * _L, _L)
        tlt_raw = tlt_v[sl]
        m = lax.shift_right_logical(tlt_raw, mshift).astype(jnp.float32)
        tl_x, tl_y = split_xy(tlt_raw & tag_mask)
        bl_x, bl_y = split_xy(blt_v[sl])
        br_x, br_y = split_xy(brt_v[sl])
        tl_xs = tl_x + val_tl0[sl]
        tl_ys = tl_y + val_tl1[sl]
        bl_xs = bl_x + val_bl0[sl]
        bl_ys = bl_y + val_bl1[sl]
        br_xs = br_x + val_br0[sl]
        br_ys = br_y + val_br1[sl]
        dx1 = bl_xs - tl_xs
        dy1 = bl_ys - tl_ys
        dx2 = bl_xs - br_xs
        dy2 = bl_ys - br_ys
        w2 = dx2 * dx2 + dy2 * dy2
        h2 = dx1 * dx1 + dy1 * dy1
        ip = dx1 * dx2 + dy1 * dy2
        acc = acc + jnp.abs(ip * _rsqrt_newton(w2 * h2)) * m
        n_acc = n_acc + m

    src_sc[pl.ds(0, _L)] = acc
    src_sc[pl.ds(_L, _L)] = n_acc
    idx_sc[pl.ds(0, _L)] = jnp.full((_L,), b, jnp.int32)
    idx_sc[pl.ds(_L, _L)] = jnp.full((_L,), _L + b, jnp.int32)
    zeros_v[pl.ds(0, _L)] = jnp.zeros((_L,), jnp.float32)
    zeros_v[pl.ds(_L, _L)] = jnp.zeros((_L,), jnp.float32)
    zeros_v[pl.ds(2 * _L, _L)] = jnp.zeros((_L,), jnp.float32)

    @pl.when(b == 0)
    def _():
        pltpu.sync_copy(zeros_v, shared)

    plsc.subcore_barrier()
    pltpu.sync_copy(src_sc, shared.at[idx_sc], add=True)
    plsc.subcore_barrier()

    @pl.when(b == 0)
    def _():
        pltpu.sync_copy(shared, sn_v)
        s_vec = sn_v[pl.ds(0, _L)]
        n_vec = sn_v[pl.ds(_L, _L)]
        loss_v[...] = s_vec / (n_vec + 0.0001)
        lidx[...] = jnp.full((_L,), 2 * _L, jnp.int32)
        pltpu.sync_copy(loss_v, shared.at[lidx], add=True)
        pltpu.sync_copy(shared.at[pl.ds(2 * _L, _L)], res_v)
        pltpu.sync_copy(res_v.at[pl.ds(0, 1)], out_hbm)


def kernel(tl_reg, bl_reg, br_reg, tl_tag, bl_tag, br_tag, mask):
    B, C, H, W = tl_reg.shape
    K = tl_tag.shape[1]
    HW = H * W
    mshift = max(HW.bit_length(), 1)

    tl_f = tl_reg.reshape(-1)
    bl_f = bl_reg.reshape(-1)
    br_f = br_reg.reshape(-1)
    tlt = (tl_tag.astype(jnp.int32)
           | (mask.astype(jnp.int32) << mshift)).reshape(-1)
    blt = bl_tag.astype(jnp.int32).reshape(-1)
    brt = br_tag.astype(jnp.int32).reshape(-1)

    mesh = plsc.VectorSubcoreMesh(core_axis_name="c", subcore_axis_name="s",
                                  num_cores=1, num_subcores=_NS)
    sc = pl.kernel(
        functools.partial(_sc_body, B, C, H, W, K, mshift),
        out_type=jax.ShapeDtypeStruct((1,), jnp.float32),
        mesh=mesh,
        scratch_types=[
            pltpu.VMEM((K,), jnp.int32),
            pltpu.VMEM((K,), jnp.int32),
            pltpu.VMEM((K,), jnp.int32),
            pltpu.VMEM((K,), jnp.int32),
            pltpu.VMEM((K,), jnp.int32),
            pltpu.VMEM((K,), jnp.int32),
            pltpu.VMEM((K,), jnp.int32),
            pltpu.VMEM((K,), jnp.int32),
            pltpu.VMEM((K,), jnp.int32),
            pltpu.VMEM((K,), jnp.float32),
            pltpu.VMEM((K,), jnp.float32),
            pltpu.VMEM((K,), jnp.float32),
            pltpu.VMEM((K,), jnp.float32),
            pltpu.VMEM((K,), jnp.float32),
            pltpu.VMEM((K,), jnp.float32),
            pltpu.VMEM((2 * _L,), jnp.float32),
            pltpu.VMEM((2 * _L,), jnp.int32),
            pltpu.VMEM((_L,), jnp.float32),
            pltpu.VMEM((3 * _L,), jnp.float32),
            pltpu.VMEM((_L,), jnp.int32),
            pltpu.VMEM((3 * _L,), jnp.float32),
            pltpu.VMEM((_L,), jnp.float32),
            pltpu.VMEM_SHARED((3 * _L,), jnp.float32),
            pltpu.SemaphoreType.DMA,
        ],
    )
    out = sc(tl_f, bl_f, br_f, tlt, blt, brt)
    return out.reshape(())


# speedup vs baseline: 2.5316x; 1.0456x over previous
import functools

import jax
import jax.numpy as jnp
from jax import lax
from jax.experimental import pallas as pl
from jax.experimental.pallas import tpu as pltpu
from jax.experimental.pallas import tpu_sc as plsc

_NS = 16
_L = 16


def _rsqrt_newton(x):
    xi = lax.bitcast_convert_type(x, jnp.int32)
    yi = jnp.int32(0x5F3759DF) - lax.shift_right_logical(xi, 1)
    y = lax.bitcast_convert_type(yi, jnp.float32)
    half_x = 0.5 * x
    for _ in range(4):
        y = y * (1.5 - half_x * y * y)
    return y


def _sc_body(B, C, H, W, K, mshift,
             tl_hbm, bl_hbm, br_hbm, tags_hbm,
             out_hbm,
             tags_v, idx6, val6, misc_f, idx_sc, lidx, shared, sem):
    HW = H * W
    CHW = C * HW
    n_chunks = K // _L
    tag_mask = (1 << mshift) - 1

    b = lax.axis_index("s")
    item_base = b * K

    pltpu.sync_copy(tags_hbm.at[pl.ds(0, 3), pl.ds(item_base, K)], tags_v)

    base_c0 = b * CHW
    for j in range(n_chunks):
        sl = pl.ds(j * _L, _L)
        t = (tags_v[0, sl] & tag_mask) + base_c0
        idx6[0, sl] = t
        idx6[1, sl] = t + HW
        t = tags_v[1, sl] + base_c0
        idx6[2, sl] = t
        idx6[3, sl] = t + HW
        t = tags_v[2, sl] + base_c0
        idx6[4, sl] = t
        idx6[5, sl] = t + HW

    copies = [
        pltpu.async_copy(tl_hbm.at[idx6.at[0]], val6.at[0], sem),
        pltpu.async_copy(tl_hbm.at[idx6.at[1]], val6.at[1], sem),
        pltpu.async_copy(bl_hbm.at[idx6.at[2]], val6.at[2], sem),
        pltpu.async_copy(bl_hbm.at[idx6.at[3]], val6.at[3], sem),
        pltpu.async_copy(br_hbm.at[idx6.at[4]], val6.at[4], sem),
        pltpu.async_copy(br_hbm.at[idx6.at[5]], val6.at[5], sem),
    ]
    for c in copies:
        c.wait()

    if W & (W - 1) == 0:
        w_shift = W.bit_length() - 1

        def split_xy(t):
            return (t & (W - 1)).astype(jnp.float32), \
                   lax.shift_right_logical(t, w_shift).astype(jnp.float32)
    else:
        def split_xy(t):
            return (t % W).astype(jnp.float32), (t // W).astype(jnp.float32)

    acc = jnp.zeros((_L,), jnp.float32)
    n_acc = jnp.zeros((_L,), jnp.float32)
    for j in range(n_chunks):
        sl = pl.ds(j * _L, _L)
        tlt_raw = tags_v[0, sl]
        m = lax.shift_right_logical(tlt_raw, mshift).astype(jnp.float32)
        tl_x, tl_y = split_xy(tlt_raw & tag_mask)
        bl_x, bl_y = split_xy(tags_v[1, sl])
        br_x, br_y = split_xy(tags_v[2, sl])
        tl_xs = tl_x + val6[0, sl]
        tl_ys = tl_y + val6[1, sl]
        bl_xs = bl_x + val6[2, sl]
        bl_ys = bl_y + val6[3, sl]
        br_xs = br_x + val6[4, sl]
        br_ys = br_y + val6[5, sl]
        dx1 = bl_xs - tl_xs
        dy1 = bl_ys - tl_ys
        dx2 = bl_xs - br_xs
        dy2 = bl_ys - br_ys
        w2 = dx2 * dx2 + dy2 * dy2
        h2 = dx1 * dx1 + dy1 * dy1
        ip = dx1 * dx2 + dy1 * dy2
        acc = acc + jnp.abs(ip * _rsqrt_newton(w2 * h2)) * m
        n_acc = n_acc + m

    zvec = jnp.zeros((_L,), jnp.float32)
    misc_f[pl.ds(0, _L)] = acc
    misc_f[pl.ds(_L, _L)] = n_acc
    misc_f[pl.ds(48, _L)] = zvec
    misc_f[pl.ds(64, _L)] = zvec
    misc_f[pl.ds(80, _L)] = zvec
    idx_sc[pl.ds(0, _L)] = jnp.full((_L,), b, jnp.int32)
    idx_sc[pl.ds(_L, _L)] = jnp.full((_L,), _L + b, jnp.int32)

    @pl.when(b == 0)
    def _():
        pltpu.sync_copy(misc_f.at[pl.ds(48, 48)], shared)

    plsc.subcore_barrier()
    pltpu.sync_copy(misc_f.at[pl.ds(0, 2 * _L)], shared.at[idx_sc], add=True)
    plsc.subcore_barrier()

    @pl.when(b == 0)
    def _():
        pltpu.sync_copy(shared, misc_f.at[pl.ds(96, 48)])
        s_vec = misc_f[pl.ds(96, _L)]
        n_vec = misc_f[pl.ds(112, _L)]
        misc_f[pl.ds(32, _L)] = s_vec / (n_vec + 0.0001)
        lidx[pl.ds(0, _L)] = jnp.full((_L,), 2 * _L, jnp.int32)
        lidx[pl.ds(_L, _L)] = jnp.full((_L,), 2 * _L + 8, jnp.int32)
        pltpu.sync_copy(misc_f.at[pl.ds(32, 2 * _L)], shared.at[lidx],
                        add=True)
        pltpu.sync_copy(shared.at[pl.ds(2 * _L, _L)], misc_f.at[pl.ds(144, _L)])
        pltpu.sync_copy(misc_f.at[pl.ds(144, 1)], out_hbm)


def kernel(tl_reg, bl_reg, br_reg, tl_tag, bl_tag, br_tag, mask):
    B, C, H, W = tl_reg.shape
    K = tl_tag.shape[1]
    HW = H * W
    mshift = max(HW.bit_length(), 1)

    tl_f = tl_reg.reshape(-1)
    bl_f = bl_reg.reshape(-1)
    br_f = br_reg.reshape(-1)
    tags = jnp.stack([
        tl_tag.astype(jnp.int32) | (mask.astype(jnp.int32) << mshift),
        bl_tag.astype(jnp.int32),
        br_tag.astype(jnp.int32),
    ]).reshape(3, B * K)

    mesh = plsc.VectorSubcoreMesh(core_axis_name="c", subcore_axis_name="s",
                                  num_cores=1, num_subcores=_NS)
    sc = pl.kernel(
        functools.partial(_sc_body, B, C, H, W, K, mshift),
        out_type=jax.ShapeDtypeStruct((1,), jnp.float32),
        mesh=mesh,
        scratch_types=[
            pltpu.VMEM((3, K), jnp.int32),
            pltpu.VMEM((6, K), jnp.int32),
            pltpu.VMEM((6, K), jnp.float32),
            pltpu.VMEM((160,), jnp.float32),
            pltpu.VMEM((2 * _L,), jnp.int32),
            pltpu.VMEM((2 * _L,), jnp.int32),
            pltpu.VMEM_SHARED((48,), jnp.float32),
            pltpu.SemaphoreType.DMA,
        ],
    )
    out = sc(tl_f, bl_f, br_f, tags)
    return out.reshape(())
